# Initial kernel scaffold; baseline (speedup 1.0000x reference)
import jax
import jax.numpy as jnp
from jax.experimental import pallas as pl


def kernel(x, edge_index, edge_weight, W1, b1, W2, b2):
    raise NotImplementedError("write your pallas kernel here")


# speedup vs baseline: 4.1891x; 4.1891x over previous
import dataclasses
import functools

import jax
import jax.numpy as jnp
from jax import lax
from jax.experimental import pallas as pl
from jax.experimental.pallas import tpu as pltpu
from jax.experimental.pallas import tpu_sc as plsc

N_NODES = 10000
N_EDGES = 320000
D_IN = 128
D_HID = 128
N_CLASSES = 64

NCORE = 2
NSUB = 16
NW = NCORE * NSUB
K = 128
NCH = (N_EDGES + NW * K - 1) // (NW * K)
E_PAD = NW * NCH * K
N_PAD = 10240
ROWS_PER_SUB = N_PAD // NSUB


def _linear1(x, W1, b1):
    def body(x_ref, w_ref, b_ref, o_ref):
        o_ref[...] = lax.dot_general(
            x_ref[...], w_ref[...], (((1,), (1,)), ((), ())),
            preferred_element_type=jnp.float32) + b_ref[...]

    return pl.pallas_call(
        body,
        grid=(10,),
        in_specs=[
            pl.BlockSpec((N_NODES // 10, D_IN), lambda i: (i, 0)),
            pl.BlockSpec((D_HID, D_IN), lambda i: (0, 0)),
            pl.BlockSpec((1, D_HID), lambda i: (0, 0)),
        ],
        out_specs=pl.BlockSpec((N_NODES // 10, D_HID), lambda i: (i, 0)),
        out_shape=jax.ShapeDtypeStruct((N_NODES, D_HID), jnp.float32),
    )(x, W1, b1)


def _relu_linear2(p, W2p, b2p):
    def body(p_ref, w_ref, b_ref, o_ref):
        h = jnp.maximum(p_ref[0] + p_ref[1], 0.0)
        o_ref[...] = lax.dot_general(
            h, w_ref[...], (((1,), (1,)), ((), ())),
            preferred_element_type=jnp.float32) + b_ref[...]

    return pl.pallas_call(
        body,
        grid=(10,),
        in_specs=[
            pl.BlockSpec((NCORE, N_NODES // 10, D_HID), lambda i: (0, i, 0)),
            pl.BlockSpec((D_HID, D_HID), lambda i: (0, 0)),
            pl.BlockSpec((1, D_HID), lambda i: (0, 0)),
        ],
        out_specs=pl.BlockSpec((N_NODES // 10, D_HID), lambda i: (i, 0)),
        out_shape=jax.ShapeDtypeStruct((N_NODES, D_HID), jnp.float32),
    )(p, W2p, b2p)


def _sum_partials(q):
    def body(q_ref, o_ref):
        o_ref[...] = q_ref[0, :, :N_CLASSES] + q_ref[1, :, :N_CLASSES]

    return pl.pallas_call(
        body,
        grid=(10,),
        in_specs=[pl.BlockSpec((NCORE, N_NODES // 10, D_HID),
                               lambda i: (0, i, 0))],
        out_specs=pl.BlockSpec((N_NODES // 10, N_CLASSES), lambda i: (i, 0)),
        out_shape=jax.ShapeDtypeStruct((N_NODES, N_CLASSES), jnp.float32),
    )(q)


def _spmm_sc(h, src, dst, w, zeros, d, d_active):
    mesh = plsc.VectorSubcoreMesh(core_axis_name="c", subcore_axis_name="s")
    cp = pltpu.CompilerParams()
    if "needs_layout_passes" in pltpu.CompilerParams.__dataclass_fields__:
        cp = dataclasses.replace(cp, needs_layout_passes=False)

    @functools.partial(
        pl.kernel,
        out_type=jax.ShapeDtypeStruct((NCORE, N_PAD, d), jnp.float32),
        mesh=mesh,
        compiler_params=cp,
        scratch_types=[
            pltpu.VMEM((NCH, K), jnp.int32),
            pltpu.VMEM((NCH, K), jnp.int32),
            pltpu.VMEM((NCH, K), jnp.float32),
            pltpu.VMEM((K, d), jnp.float32),
            pltpu.VMEM_SHARED((N_PAD, d), jnp.float32),
            pltpu.SemaphoreType.DMA,
        ],
    )
    def k(h_hbm, src_hbm, dst_hbm, w_hbm, z_hbm, out_hbm,
          src_vm, dst_vm, w_vm, rows_vm, acc_sh, sem):
        cid = lax.axis_index("c")
        sid = lax.axis_index("s")
        wid = cid * NSUB + sid

        pltpu.sync_copy(z_hbm.at[pl.ds(sid * ROWS_PER_SUB, ROWS_PER_SUB)],
                        acc_sh.at[pl.ds(sid * ROWS_PER_SUB, ROWS_PER_SUB)])

        pltpu.sync_copy(src_hbm.at[wid], src_vm)
        pltpu.sync_copy(dst_hbm.at[wid], dst_vm)
        pltpu.sync_copy(w_hbm.at[wid], w_vm)

        plsc.subcore_barrier()

        @pl.loop(0, NCH)
        def _(j):
            pltpu.async_copy(h_hbm.at[src_vm.at[j]], rows_vm, sem).wait()

            @pl.loop(0, K)
            def _(e):
                jj = jnp.full((16,), j, jnp.int32)
                ee = jnp.full((16,), e, jnp.int32)
                wsplat = plsc.load_gather(w_vm, [jj, ee])
                for g in range(d_active // 16):
                    sl = (e, pl.ds(g * 16, 16))
                    rows_vm[sl] = rows_vm[sl] * wsplat

            pltpu.sync_copy(rows_vm, acc_sh.at[dst_vm.at[j]], add=True)

        plsc.subcore_barrier()

        pltpu.sync_copy(
            acc_sh.at[pl.ds(sid * ROWS_PER_SUB, ROWS_PER_SUB)],
            out_hbm.at[cid].at[pl.ds(sid * ROWS_PER_SUB, ROWS_PER_SUB)])

    return k(h, src, dst, w, zeros)


def kernel(x, edge_index, edge_weight, W1, b1, W2, b2):
    pad = E_PAD - N_EDGES
    dst = jnp.pad(edge_index[0].astype(jnp.int32), (0, pad)).reshape(NW, NCH, K)
    src = jnp.pad(edge_index[1].astype(jnp.int32), (0, pad)).reshape(NW, NCH, K)
    w = jnp.pad(edge_weight, (0, pad)).reshape(NW, NCH, K)
    z1 = jnp.zeros((N_PAD, D_HID), jnp.float32)
    W2p = jnp.pad(W2, ((0, D_HID - N_CLASSES), (0, 0)))
    b2p = jnp.pad(b2, (0, D_HID - N_CLASSES)).reshape(1, D_HID)

    h = _linear1(x, W1, b1.reshape(1, D_HID))
    p = _spmm_sc(h, src, dst, w, z1, D_HID, D_HID)
    h2 = _relu_linear2(p, W2p, b2p)
    q = _spmm_sc(h2, src, dst, w, z1, D_HID, N_CLASSES)
    return _sum_partials(q)
